# baseline (speedup 1.0000x reference)
import jax
import jax.numpy as jnp
from jax import lax
from jax.experimental import pallas as pl
from jax.experimental.pallas import tpu as pltpu
from jax.experimental.pallas import tpu_sc as plsc

_E_GEN = 320000
_E_GIVEN = 320000
_NC = 2
_NS = 16
_NW = _NC * _NS
_CHUNK = _E_GEN // _NW


def _concat_body(gen_s, gen_t, gen_w, giv_s, giv_t,
                 out_s, out_t, out_w,
                 b_gs, b_givs, b_gt, b_givt, b_gw, ones_v,
                 in_sems, out_sems):
    wid = lax.axis_index("s") * _NC + lax.axis_index("c")
    base = wid * _CHUNK
    src = pl.ds(base, _CHUNK)
    lo = pl.ds(base, _CHUNK)
    hi = pl.ds(_E_GEN + base, _CHUNK)

    moves = (
        (gen_s, b_gs, out_s, lo),
        (giv_s, b_givs, out_s, hi),
        (gen_t, b_gt, out_t, lo),
        (giv_t, b_givt, out_t, hi),
        (gen_w, b_gw, out_w, lo),
    )
    in_copies = [
        pltpu.async_copy(inp.at[src], buf, in_sems.at[i])
        for i, (inp, buf, _, _) in enumerate(moves)
    ]

    def _fill(i, carry):
        ones_v[pl.ds(i * 16, 16)] = jnp.full((16,), 1.0, jnp.float32)
        return carry
    lax.fori_loop(0, _CHUNK // 16, _fill, 0)
    out_copies = [pltpu.async_copy(ones_v, out_w.at[hi], out_sems.at[5])]

    for i, (_, buf, out, dst) in enumerate(moves):
        in_copies[i].wait()
        out_copies.append(pltpu.async_copy(buf, out.at[dst], out_sems.at[i]))

    for c in out_copies:
        c.wait()


@jax.jit
def _concat_sc(gen_s, gen_t, gen_w, giv_s, giv_t):
    n_out = _E_GEN + _E_GIVEN
    run = pl.kernel(
        _concat_body,
        out_type=(
            jax.ShapeDtypeStruct((n_out,), jnp.int32),
            jax.ShapeDtypeStruct((n_out,), jnp.int32),
            jax.ShapeDtypeStruct((n_out,), jnp.float32),
        ),
        mesh=plsc.VectorSubcoreMesh(core_axis_name="c", subcore_axis_name="s"),
        scratch_types=[
            pltpu.VMEM((_CHUNK,), jnp.int32),
            pltpu.VMEM((_CHUNK,), jnp.int32),
            pltpu.VMEM((_CHUNK,), jnp.int32),
            pltpu.VMEM((_CHUNK,), jnp.int32),
            pltpu.VMEM((_CHUNK,), jnp.float32),
            pltpu.VMEM((_CHUNK,), jnp.float32),
            pltpu.SemaphoreType.DMA((5,)),
            pltpu.SemaphoreType.DMA((6,)),
        ],
    )
    return run(gen_s, gen_t, gen_w, giv_s, giv_t)


def kernel(gen_sources, gen_targets, gen_weights, given_sources,
           given_targets, node_embeddings):
    noisy_sources, noisy_targets, edge_weights = _concat_sc(
        gen_sources, gen_targets, gen_weights, given_sources, given_targets)
    return noisy_sources, noisy_targets, edge_weights, node_embeddings
